# baseline (speedup 1.0000x reference)
import functools

import numpy as np
import jax
import jax.numpy as jnp
from jax import lax
from jax.experimental import pallas as pl
from jax.experimental.pallas import tpu as pltpu
from jax.experimental.pallas import tpu_sc as plsc

NUM_LOCATIONS = 1000000
BASE_DIM = 64
BANDWIDTH = 0.5
BATCH = 4096
SEQ_LEN = 50

NUM_CORES = 2
NUM_SUBCORES = 16
NW = NUM_CORES * NUM_SUBCORES
BLK = BATCH // NW

_t = np.arange(24, dtype=np.float32)
_absdiff = np.abs(_t[None, :] - _t[:, None])
_dist = np.minimum(_absdiff, 24.0 - _absdiff)
_W_SMOOTH = np.exp(-0.5 * (_dist / BANDWIDTH) ** 2).astype(np.float32)

_sc_mesh = plsc.VectorSubcoreMesh(core_axis_name="c", subcore_axis_name="s")


@functools.partial(
    pl.kernel,
    out_type=jax.ShapeDtypeStruct((SEQ_LEN, NW, BLK, 128), jnp.float32),
    mesh=_sc_mesh,
    scratch_types=[
        pltpu.VMEM((SEQ_LEN, BLK), jnp.int32),
        pltpu.VMEM((4, BLK, 128), jnp.float32),
        pltpu.SemaphoreType.DMA,
        pltpu.SemaphoreType.DMA,
        pltpu.SemaphoreType.DMA,
        pltpu.SemaphoreType.DMA,
        pltpu.SemaphoreType.DMA,
        pltpu.SemaphoreType.DMA,
        pltpu.SemaphoreType.DMA,
        pltpu.SemaphoreType.DMA,
    ],
    compiler_params=pltpu.CompilerParams(needs_layout_passes=False),
)
def _sc_gather(idx_hbm, tbl_hbm, out_hbm, idx_v, rows_v,
               g0, g1, g2, g3, o0, o1, o2, o3):
    gsems = (g0, g1, g2, g3)
    osems = (o0, o1, o2, o3)
    wid = lax.axis_index("s") * NUM_CORES + lax.axis_index("c")
    b0 = wid * BLK
    pltpu.sync_copy(idx_hbm.at[:, pl.ds(b0, BLK)], idx_v)

    def start_gather(s, slot):
        pltpu.async_copy(tbl_hbm.at[idx_v.at[s]], rows_v.at[slot], gsems[slot])

    def wait_gather(s, slot):
        pltpu.make_async_copy(tbl_hbm.at[idx_v.at[s]], rows_v.at[slot],
                              gsems[slot]).wait()

    def start_write(s, slot):
        pltpu.async_copy(rows_v.at[slot], out_hbm.at[s, wid], osems[slot])

    def wait_write(s, slot):
        pltpu.make_async_copy(rows_v.at[slot], out_hbm.at[s, wid],
                              osems[slot]).wait()

    for s in range(4):
        start_gather(s, s)

    def quad_body(i, _):
        c0 = i * 4
        for k in range(4):
            c = c0 + k

            @pl.when(c < SEQ_LEN)
            def _():
                wait_gather(c, k)
                start_write(c, k)

                @pl.when(c + 4 < SEQ_LEN)
                def _():
                    wait_write(c, k)
                    start_gather(c + 4, k)
        return ()

    lax.fori_loop(0, (SEQ_LEN + 3) // 4, quad_body, ())
    wait_write(SEQ_LEN - 4, 2)
    wait_write(SEQ_LEN - 3, 3)
    wait_write(SEQ_LEN - 2, 0)
    wait_write(SEQ_LEN - 1, 1)


def _smooth_body(w_ref, t_ref, o_ref):
    o_ref[...] = jnp.dot(w_ref[...], t_ref[...],
                         preferred_element_type=jnp.float32)


def _transpose_body(i_ref, o_ref):
    o_ref[0] = jnp.swapaxes(i_ref[0, 0, :, :BASE_DIM], 0, 1)


def kernel(location_x, loc_table, user_table, time_table):
    idx_t = location_x.astype(jnp.int32).T
    tbl_pad = jnp.pad(loc_table, ((0, 0), (0, BASE_DIM)))
    rows = _sc_gather(idx_t, tbl_pad)
    out_t = pl.pallas_call(
        _transpose_body,
        grid=(SEQ_LEN, NW),
        in_specs=[pl.BlockSpec((1, 1, BLK, 128), lambda s, b: (s, b, 0, 0))],
        out_specs=pl.BlockSpec((1, BASE_DIM, BLK), lambda s, b: (s, 0, b)),
        out_shape=jax.ShapeDtypeStruct((SEQ_LEN, BASE_DIM, BATCH),
                                       jnp.float32),
    )(rows)
    loc_embedded = out_t.transpose(2, 0, 1)
    smoothed = pl.pallas_call(
        _smooth_body,
        out_shape=jax.ShapeDtypeStruct((24, BASE_DIM), jnp.float32),
    )(jnp.asarray(_W_SMOOTH), time_table)
    return (loc_embedded, time_table, smoothed, user_table)


# speedup vs baseline: 1.7924x; 1.7924x over previous
import functools

import numpy as np
import jax
import jax.numpy as jnp
from jax import lax
from jax.experimental import pallas as pl
from jax.experimental.pallas import tpu as pltpu
from jax.experimental.pallas import tpu_sc as plsc

NUM_LOCATIONS = 1000000
BASE_DIM = 64
BANDWIDTH = 0.5
BATCH = 4096
SEQ_LEN = 50

NUM_CORES = 2
NUM_SUBCORES = 16
NW = NUM_CORES * NUM_SUBCORES
BLK = BATCH // NW

_t = np.arange(24, dtype=np.float32)
_absdiff = np.abs(_t[None, :] - _t[:, None])
_dist = np.minimum(_absdiff, 24.0 - _absdiff)
_W_SMOOTH = np.exp(-0.5 * (_dist / BANDWIDTH) ** 2).astype(np.float32)

_sc_mesh = plsc.VectorSubcoreMesh(core_axis_name="c", subcore_axis_name="s")


@functools.partial(
    pl.kernel,
    out_type=jax.ShapeDtypeStruct((SEQ_LEN, BASE_DIM, BATCH), jnp.float32),
    mesh=_sc_mesh,
    scratch_types=[
        pltpu.VMEM((SEQ_LEN, BLK), jnp.int32),
        pltpu.VMEM((4, BLK, 128), jnp.float32),
        pltpu.VMEM((4, BASE_DIM, BLK), jnp.float32),
        pltpu.SemaphoreType.DMA,
        pltpu.SemaphoreType.DMA,
        pltpu.SemaphoreType.DMA,
        pltpu.SemaphoreType.DMA,
        pltpu.SemaphoreType.DMA,
        pltpu.SemaphoreType.DMA,
        pltpu.SemaphoreType.DMA,
        pltpu.SemaphoreType.DMA,
    ],
    compiler_params=pltpu.CompilerParams(needs_layout_passes=False),
)
def _sc_gather(idx_hbm, tbl_hbm, out_hbm, idx_v, rows_v, outt_v,
               g0, g1, g2, g3, o0, o1, o2, o3):
    gsems = (g0, g1, g2, g3)
    osems = (o0, o1, o2, o3)
    wid = lax.axis_index("s") * NUM_CORES + lax.axis_index("c")
    b0 = wid * BLK
    pltpu.sync_copy(idx_hbm.at[:, pl.ds(b0, BLK)], idx_v)

    lanes = lax.iota(jnp.int32, 16)

    def start_gather(s, slot):
        pltpu.async_copy(tbl_hbm.at[idx_v.at[s]], rows_v.at[slot], gsems[slot])

    def wait_gather(s, slot):
        pltpu.make_async_copy(tbl_hbm.at[idx_v.at[s]], rows_v.at[slot],
                              gsems[slot]).wait()

    def transpose_block(slot):
        rows = rows_v.at[slot]
        out = outt_v.at[slot]

        def dloop(d, _):
            for g4 in range(BLK // 64):
                vals = [plsc.load_gather(
                            rows, [lanes + (64 * g4 + 16 * k),
                                   jnp.full((16,), d, dtype=jnp.int32)])
                        for k in range(4)]
                for k in range(4):
                    out[d, pl.ds(64 * g4 + 16 * k, 16)] = vals[k]
            return ()

        lax.fori_loop(0, BASE_DIM, dloop, ())

    def start_write(s, slot):
        pltpu.async_copy(outt_v.at[slot], out_hbm.at[s, :, pl.ds(b0, BLK)],
                         osems[slot])

    def wait_write(s, slot):
        pltpu.make_async_copy(outt_v.at[slot], out_hbm.at[s, :, pl.ds(b0, BLK)],
                              osems[slot]).wait()

    for s in range(4):
        start_gather(s, s)

    def quad_body(i, _):
        c0 = i * 4
        for k in range(4):
            c = c0 + k

            @pl.when(c < SEQ_LEN)
            def _():
                wait_gather(c, k)

                @pl.when(c >= 4)
                def _():
                    wait_write(c - 4, k)
                transpose_block(k)
                start_write(c, k)

                @pl.when(c + 4 < SEQ_LEN)
                def _():
                    start_gather(c + 4, k)
        return ()

    lax.fori_loop(0, (SEQ_LEN + 3) // 4, quad_body, ())
    wait_write(SEQ_LEN - 4, 2)
    wait_write(SEQ_LEN - 3, 3)
    wait_write(SEQ_LEN - 2, 0)
    wait_write(SEQ_LEN - 1, 1)


def _smooth_body(w_ref, t_ref, o_ref):
    o_ref[...] = jnp.dot(w_ref[...], t_ref[...],
                         preferred_element_type=jnp.float32)


def kernel(location_x, loc_table, user_table, time_table):
    idx_t = location_x.astype(jnp.int32).T
    tbl_pad = jnp.pad(loc_table, ((0, 0), (0, BASE_DIM)))
    out_t = _sc_gather(idx_t, tbl_pad)
    loc_embedded = out_t.transpose(2, 0, 1)
    smoothed = pl.pallas_call(
        _smooth_body,
        out_shape=jax.ShapeDtypeStruct((24, BASE_DIM), jnp.float32),
    )(jnp.asarray(_W_SMOOTH), time_table)
    return (loc_embedded, time_table, smoothed, user_table)
